# baseline (speedup 1.0000x reference)
import functools

import jax
import jax.numpy as jnp
from jax import lax
from jax.experimental import pallas as pl
from jax.experimental.pallas import tpu as pltpu
from jax.experimental.pallas import tpu_sc as plsc


B, S, D = 4, 8192, 1024
S_BLK = 512
LANES = 16
NC, NS = 2, 16
NW = NC * NS
SC_ROWS = S
ROWS_PER_W = SC_ROWS // NW
R = 16

_mesh = plsc.VectorSubcoreMesh(core_axis_name="c", subcore_axis_name="s")


@functools.partial(
    pl.kernel,
    out_type=jax.ShapeDtypeStruct((S * D,), jnp.float32),
    mesh=_mesh,
    scratch_types=[
        pltpu.VMEM((R * D,), jnp.float32),
        pltpu.VMEM((R * D,), jnp.float32),
    ],
)
def _sc_add_row3(x_hbm, te_hbm, o_hbm, x_v, te_v):
    wid = lax.axis_index("s") * NC + lax.axis_index("c")
    row0 = wid * ROWS_PER_W

    def chunk(c, carry):
        sbase = (row0 + c * R) * D
        pltpu.sync_copy(x_hbm.at[pl.ds(3 * S * D + sbase, R * D)], x_v)
        pltpu.sync_copy(te_hbm.at[pl.ds(sbase, R * D)], te_v)

        @plsc.parallel_loop(0, R * D, step=LANES, unroll=8)
        def _add(i):
            sl = pl.ds(i, LANES)
            x_v[sl] = x_v[sl] + te_v[sl]

        pltpu.sync_copy(x_v, o_hbm.at[pl.ds(sbase, R * D)])
        return carry

    lax.fori_loop(0, ROWS_PER_W // R, chunk, None)


def _add_pos_kernel(x_ref, pos_ref, o_ref):
    o_ref[...] = x_ref[...] + pos_ref[...][None, :, :]


def _tc_rows(x, time_embed, nb):
    return pl.pallas_call(
        _add_pos_kernel,
        grid=(S // S_BLK,),
        in_specs=[
            pl.BlockSpec((nb, S_BLK, D), lambda i: (0, i, 0)),
            pl.BlockSpec((S_BLK, D), lambda i: (i, 0)),
        ],
        out_specs=pl.BlockSpec((nb, S_BLK, D), lambda i: (0, i, 0)),
        out_shape=jax.ShapeDtypeStruct((nb, S, D), x.dtype),
    )(x, time_embed)


def kernel(x, time_embed):
    sc_out = _sc_add_row3(x.reshape(-1), time_embed.reshape(-1))
    tc_out = _tc_rows(x, time_embed, 3)
    return jnp.concatenate([tc_out, sc_out.reshape(1, S, D)], axis=0)


# speedup vs baseline: 1.8199x; 1.8199x over previous
import functools

import jax
import jax.numpy as jnp
from jax import lax
from jax.experimental import pallas as pl
from jax.experimental.pallas import tpu as pltpu
from jax.experimental.pallas import tpu_sc as plsc


B, S, D = 4, 8192, 1024
S_BLK = 512
LANES = 16
NC, NS = 2, 16
NW = NC * NS
SC_BATCH = 3
ROWS_PER_W = S // NW
R = 16
N_CHUNKS = ROWS_PER_W // R

_mesh = plsc.VectorSubcoreMesh(core_axis_name="c", subcore_axis_name="s")


@functools.partial(
    pl.kernel,
    out_type=jax.ShapeDtypeStruct((S, D), jnp.float32),
    mesh=_mesh,
    scratch_types=[
        pltpu.VMEM((R, D), jnp.float32),
        pltpu.VMEM((R, D), jnp.float32),
        pltpu.VMEM((R, D), jnp.float32),
        pltpu.VMEM((R, D), jnp.float32),
        pltpu.VMEM((R, D), jnp.float32),
        pltpu.VMEM((R, D), jnp.float32),
        pltpu.SemaphoreType.DMA,
        pltpu.SemaphoreType.DMA,
        pltpu.SemaphoreType.DMA,
        pltpu.SemaphoreType.DMA,
        pltpu.SemaphoreType.DMA,
        pltpu.SemaphoreType.DMA,
    ],
)
def _sc_add_row3(x_hbm, te_hbm, o_hbm, xv0, xv1, tv0, tv1, ov0, ov1,
                 sx0, sx1, st0, st1, so0, so1):
    wid = lax.axis_index("s") * NC + lax.axis_index("c")
    row0 = wid * ROWS_PER_W
    xv = (xv0, xv1)
    tv = (tv0, tv1)
    ov = (ov0, ov1)
    sx = (sx0, sx1)
    st = (st0, st1)
    so = (so0, so1)

    def fire_loads(c, p):
        base = row0 + c * R
        lx = pltpu.async_copy(x_hbm.at[SC_BATCH, pl.ds(base, R), :], xv[p], sx[p])
        lt = pltpu.async_copy(te_hbm.at[pl.ds(base, R), :], tv[p], st[p])
        return lx, lt

    loads = [fire_loads(0, 0), fire_loads(1, 1)]
    stores = [None, None]
    for c in range(N_CHUNKS):
        p = c % 2
        if stores[p] is not None:
            stores[p].wait()
        lx, lt = loads[p]
        lx.wait()
        lt.wait()

        @plsc.parallel_loop(0, D, step=LANES, unroll=2)
        def _add(i):
            sl = pl.ds(i, LANES)
            for r in range(R):
                ov[p][r, sl] = xv[p][r, sl] + tv[p][r, sl]

        if c + 2 < N_CHUNKS:
            loads[p] = fire_loads(c + 2, p)
        base = row0 + c * R
        stores[p] = pltpu.async_copy(ov[p], o_hbm.at[pl.ds(base, R), :], so[p])
    stores[0].wait()
    stores[1].wait()


def _add_pos_kernel(x_ref, pos_ref, o_ref):
    o_ref[...] = x_ref[...] + pos_ref[...][None, :, :]


def _tc_rows03(x, time_embed):
    return pl.pallas_call(
        _add_pos_kernel,
        grid=(S // S_BLK,),
        in_specs=[
            pl.BlockSpec((SC_BATCH, S_BLK, D), lambda i: (0, i, 0)),
            pl.BlockSpec((S_BLK, D), lambda i: (i, 0)),
        ],
        out_specs=pl.BlockSpec((SC_BATCH, S_BLK, D), lambda i: (0, i, 0)),
        out_shape=jax.ShapeDtypeStruct((SC_BATCH, S, D), x.dtype),
    )(x, time_embed)


def kernel(x, time_embed):
    sc_out = _sc_add_row3(x, time_embed)
    tc_out = _tc_rows03(x, time_embed)
    return jnp.concatenate([tc_out, sc_out[None]], axis=0)


# speedup vs baseline: 4.0166x; 2.2070x over previous
import jax
import jax.numpy as jnp
from jax.experimental import pallas as pl


S_BLK = 512


def _add_pos_kernel(x_ref, pos_ref, o_ref):
    o_ref[...] = x_ref[...] + pos_ref[...][None, :, :]


def kernel(x, time_embed):
    B, S, d = x.shape
    grid = (S // S_BLK,)
    return pl.pallas_call(
        _add_pos_kernel,
        grid=grid,
        in_specs=[
            pl.BlockSpec((B, S_BLK, d), lambda i: (0, i, 0)),
            pl.BlockSpec((S_BLK, d), lambda i: (i, 0)),
        ],
        out_specs=pl.BlockSpec((B, S_BLK, d), lambda i: (0, i, 0)),
        out_shape=jax.ShapeDtypeStruct((B, S, d), x.dtype),
    )(x, time_embed)
